# baseline (speedup 1.0000x reference)
import jax
import jax.numpy as jnp
from jax.experimental import pallas as pl
from jax.experimental.pallas import tpu as pltpu

ROW_BLOCK = 400


def _kgagg_body(a_ref, e_ref, wsT_ref, wnT_ref, b_ref, out_ref):
    i = pl.program_id(0)
    neigh = jnp.dot(a_ref[...], e_ref[...], preferred_element_type=jnp.float32)
    neigh = jnp.dot(neigh, wnT_ref[...], preferred_element_type=jnp.float32)
    e_blk = e_ref[pl.ds(i * ROW_BLOCK, ROW_BLOCK), :]
    self_t = jnp.dot(e_blk, wsT_ref[...], preferred_element_type=jnp.float32)
    x = self_t + neigh + b_ref[...]
    out_ref[...] = jnp.where(x >= 0.0, x, 0.01 * x)


def kernel(entity_embs, adj_matrix, W_self, b_self, W_neigh, b_neigh):
    n, d_in = entity_embs.shape
    d_out = W_self.shape[0]
    bias = (b_self + b_neigh).reshape(1, d_out)
    return pl.pallas_call(
        _kgagg_body,
        grid=(n // ROW_BLOCK,),
        in_specs=[
            pl.BlockSpec((ROW_BLOCK, n), lambda i: (i, 0)),
            pl.BlockSpec((n, d_in), lambda i: (0, 0)),
            pl.BlockSpec((d_in, d_out), lambda i: (0, 0)),
            pl.BlockSpec((d_in, d_out), lambda i: (0, 0)),
            pl.BlockSpec((1, d_out), lambda i: (0, 0)),
        ],
        out_specs=pl.BlockSpec((ROW_BLOCK, d_out), lambda i: (i, 0)),
        out_shape=jax.ShapeDtypeStruct((n, d_out), jnp.float32),
        compiler_params=pltpu.CompilerParams(
            dimension_semantics=("parallel",),
        ),
    )(adj_matrix, entity_embs, W_self.T, W_neigh.T, bias)


# speedup vs baseline: 1.0018x; 1.0018x over previous
import jax
import jax.numpy as jnp
from jax.experimental import pallas as pl
from jax.experimental.pallas import tpu as pltpu

HALF_BLOCK = 200


def _kgagg_body(a0_ref, a1_ref, e_ref, wsT_ref, wnT_ref, b_ref, out_ref):
    i = pl.program_id(0)
    e = e_ref[...]
    wnT = wnT_ref[...]
    n0 = jnp.dot(a0_ref[...], e, preferred_element_type=jnp.float32)
    n1 = jnp.dot(a1_ref[...], e, preferred_element_type=jnp.float32)
    neigh = jnp.dot(jnp.concatenate([n0, n1], axis=0), wnT,
                    preferred_element_type=jnp.float32)
    e_blk = e_ref[pl.ds(i * 2 * HALF_BLOCK, 2 * HALF_BLOCK), :]
    self_t = jnp.dot(e_blk, wsT_ref[...], preferred_element_type=jnp.float32)
    x = self_t + neigh + b_ref[...]
    out_ref[...] = jnp.where(x >= 0.0, x, 0.01 * x)


def kernel(entity_embs, adj_matrix, W_self, b_self, W_neigh, b_neigh):
    n, d_in = entity_embs.shape
    d_out = W_self.shape[0]
    bias = (b_self + b_neigh).reshape(1, d_out)
    return pl.pallas_call(
        _kgagg_body,
        grid=(n // (2 * HALF_BLOCK),),
        in_specs=[
            pl.BlockSpec((HALF_BLOCK, n), lambda i: (2 * i, 0)),
            pl.BlockSpec((HALF_BLOCK, n), lambda i: (2 * i + 1, 0)),
            pl.BlockSpec((n, d_in), lambda i: (0, 0)),
            pl.BlockSpec((d_in, d_out), lambda i: (0, 0)),
            pl.BlockSpec((d_in, d_out), lambda i: (0, 0)),
            pl.BlockSpec((1, d_out), lambda i: (0, 0)),
        ],
        out_specs=pl.BlockSpec((2 * HALF_BLOCK, d_out), lambda i: (i, 0)),
        out_shape=jax.ShapeDtypeStruct((n, d_out), jnp.float32),
        compiler_params=pltpu.CompilerParams(
            dimension_semantics=("parallel",),
        ),
    )(adj_matrix, adj_matrix, entity_embs, W_self.T, W_neigh.T, bias)


# speedup vs baseline: 1.0327x; 1.0308x over previous
import jax
import jax.numpy as jnp
from jax.experimental import pallas as pl
from jax.experimental.pallas import tpu as pltpu

HALF_BLOCK = 200


def _dot_bt(x, w):
    return jax.lax.dot_general(
        x, w, dimension_numbers=(((1,), (1,)), ((), ())),
        preferred_element_type=jnp.float32)


def _kgagg_body(a0_ref, a1_ref, e_ref, ws_ref, wn_ref, bs_ref, bn_ref, out_ref):
    i = pl.program_id(0)
    e = e_ref[...]
    n0 = jnp.dot(a0_ref[...], e, preferred_element_type=jnp.float32)
    n1 = jnp.dot(a1_ref[...], e, preferred_element_type=jnp.float32)
    neigh = _dot_bt(jnp.concatenate([n0, n1], axis=0), wn_ref[...])
    e_blk = e_ref[pl.ds(i * 2 * HALF_BLOCK, 2 * HALF_BLOCK), :]
    self_t = _dot_bt(e_blk, ws_ref[...])
    x = self_t + neigh + (bs_ref[...] + bn_ref[...])
    out_ref[...] = jnp.where(x >= 0.0, x, 0.01 * x)


def kernel(entity_embs, adj_matrix, W_self, b_self, W_neigh, b_neigh):
    n, d_in = entity_embs.shape
    d_out = W_self.shape[0]
    return pl.pallas_call(
        _kgagg_body,
        grid=(n // (2 * HALF_BLOCK),),
        in_specs=[
            pl.BlockSpec((HALF_BLOCK, n), lambda i: (2 * i, 0)),
            pl.BlockSpec((HALF_BLOCK, n), lambda i: (2 * i + 1, 0)),
            pl.BlockSpec((n, d_in), lambda i: (0, 0)),
            pl.BlockSpec((d_out, d_in), lambda i: (0, 0)),
            pl.BlockSpec((d_out, d_in), lambda i: (0, 0)),
            pl.BlockSpec((1, d_out), lambda i: (0, 0)),
            pl.BlockSpec((1, d_out), lambda i: (0, 0)),
        ],
        out_specs=pl.BlockSpec((2 * HALF_BLOCK, d_out), lambda i: (i, 0)),
        out_shape=jax.ShapeDtypeStruct((n, d_out), jnp.float32),
        compiler_params=pltpu.CompilerParams(
            dimension_semantics=("parallel",),
        ),
    )(adj_matrix, adj_matrix, entity_embs, W_self, W_neigh,
      b_self.reshape(1, d_out), b_neigh.reshape(1, d_out))


# speedup vs baseline: 1.0380x; 1.0051x over previous
import jax
import jax.numpy as jnp
from jax.experimental import pallas as pl
from jax.experimental.pallas import tpu as pltpu

ROW_BLOCK = 400


def _dot_bt(x, w):
    return jax.lax.dot_general(
        x, w, dimension_numbers=(((1,), (1,)), ((), ())),
        preferred_element_type=jnp.float32)


def _kgagg_body(a_ref, e_ref, ws_ref, wn_ref, bs_ref, bn_ref, out_ref):
    i = pl.program_id(0)
    neigh = jnp.dot(a_ref[...], e_ref[...], preferred_element_type=jnp.float32)
    neigh = _dot_bt(neigh, wn_ref[...])
    e_blk = e_ref[pl.ds(i * ROW_BLOCK, ROW_BLOCK), :]
    self_t = _dot_bt(e_blk, ws_ref[...])
    x = self_t + neigh + (bs_ref[...] + bn_ref[...])
    out_ref[...] = jnp.where(x >= 0.0, x, 0.01 * x)


def kernel(entity_embs, adj_matrix, W_self, b_self, W_neigh, b_neigh):
    n, d_in = entity_embs.shape
    d_out = W_self.shape[0]
    return pl.pallas_call(
        _kgagg_body,
        grid=(n // ROW_BLOCK,),
        in_specs=[
            pl.BlockSpec((ROW_BLOCK, n), lambda i: (i, 0)),
            pl.BlockSpec((n, d_in), lambda i: (0, 0)),
            pl.BlockSpec((d_out, d_in), lambda i: (0, 0)),
            pl.BlockSpec((d_out, d_in), lambda i: (0, 0)),
            pl.BlockSpec((1, d_out), lambda i: (0, 0)),
            pl.BlockSpec((1, d_out), lambda i: (0, 0)),
        ],
        out_specs=pl.BlockSpec((ROW_BLOCK, d_out), lambda i: (i, 0)),
        out_shape=jax.ShapeDtypeStruct((n, d_out), jnp.float32),
        compiler_params=pltpu.CompilerParams(
            dimension_semantics=("parallel",),
        ),
    )(adj_matrix, entity_embs, W_self, W_neigh,
      b_self.reshape(1, d_out), b_neigh.reshape(1, d_out))
